# baseline (speedup 1.0000x reference)
import functools

import jax
import jax.numpy as jnp
from jax import lax
from jax.experimental import pallas as pl
from jax.experimental.pallas import tpu as pltpu
from jax.experimental.pallas import tpu_sc as plsc

VOCAB = 1000000
INPUT_DIM = 64
BATCH = 4096
HIST = 200

_NW = 32
_BBLK = BATCH // _NW
_NBUF = 2
_K = 4
_NGRP = HIST // _K


def _gather_kernel(table_hbm, xt_hbm, out_hbm, idx_v, rows_v, trows_v,
                   gsem, osem):
    nc = 2
    wid = lax.axis_index("s") * nc + lax.axis_index("c")
    b0 = wid * _BBLK

    pltpu.sync_copy(xt_hbm.at[:, pl.ds(b0, _BBLK)], idx_v)

    row_iotas = [lax.iota(jnp.int32, 16) + 16 * g for g in range(8)]

    def gather_start(t, b):
        pltpu.async_copy(table_hbm.at[idx_v.at[t]], rows_v.at[b], gsem.at[b])

    def gather_wait(b):
        pltpu.make_async_copy(
            table_hbm.at[pl.ds(0, _BBLK)], rows_v.at[b], gsem.at[b]
        ).wait()

    def out_start(g, gb):
        pltpu.async_copy(
            trows_v.at[gb],
            out_hbm.at[pl.ds(g * _K, _K), :, pl.ds(b0, _BBLK)],
            osem.at[gb],
        )

    def out_wait(gb):
        pltpu.make_async_copy(
            trows_v.at[gb],
            out_hbm.at[pl.ds(0, _K), :, pl.ds(0, _BBLK)],
            osem.at[gb],
        ).wait()

    def transpose_unit(b, gb, k):
        @plsc.parallel_loop(0, INPUT_DIM, 1, unroll=8)
        def _(d):
            col = jnp.full((16,), jnp.int32(0)) + d
            for gi in range(8):
                v = plsc.load_gather(rows_v.at[b], [row_iotas[gi], col])
                trows_v[gb, k, d, pl.ds(16 * gi, 16)] = v

    for b in range(_NBUF):
        gather_start(b, b)

    def body(g, carry):
        gb = lax.rem(g, 2)

        @pl.when(g > 1)
        def _():
            out_wait(gb)

        for k in range(_K):
            t = g * _K + k
            b = k % _NBUF
            gather_wait(b)
            transpose_unit(b, gb, k)

            @pl.when(t + _NBUF < HIST)
            def _():
                gather_start(t + _NBUF, b)

        out_start(g, gb)
        return carry

    lax.fori_loop(0, _NGRP, body, 0)

    for gb in range(2):
        out_wait(gb)


@jax.jit
def kernel(x, emb_table):
    xt = x.T
    run = functools.partial(
        pl.kernel,
        mesh=plsc.VectorSubcoreMesh(core_axis_name="c", subcore_axis_name="s"),
        out_type=jax.ShapeDtypeStruct((HIST, INPUT_DIM, BATCH), jnp.float32),
        scratch_types=[
            pltpu.VMEM((HIST, _BBLK), jnp.int32),
            pltpu.VMEM((_NBUF, _BBLK, INPUT_DIM), jnp.float32),
            pltpu.VMEM((2, _K, INPUT_DIM, _BBLK), jnp.float32),
            pltpu.SemaphoreType.DMA((_NBUF,)),
            pltpu.SemaphoreType.DMA((2,)),
        ],
        compiler_params=pltpu.CompilerParams(use_tc_tiling_on_sc=False,
                                             needs_layout_passes=False),
    )(_gather_kernel)
    out_t = run(emb_table, xt)
    return jnp.transpose(out_t, (2, 0, 1))


# speedup vs baseline: 1.0023x; 1.0023x over previous
import functools

import jax
import jax.numpy as jnp
from jax import lax
from jax.experimental import pallas as pl
from jax.experimental.pallas import tpu as pltpu
from jax.experimental.pallas import tpu_sc as plsc

VOCAB = 1000000
INPUT_DIM = 64
BATCH = 4096
HIST = 200

_BBLK = 512
_NB = BATCH // _BBLK
_TBLK = 50
_CHUNK = 128
_NC = _BBLK // _CHUNK
_NBUF = 2


def _gather_kernel(table_hbm, xt_hbm, out_hbm, idx_v, rows_v, trows_v,
                   gsem, osem):
    nc = 2
    wid = lax.axis_index("s") * nc + lax.axis_index("c")
    tr = wid // _NB
    wb = wid % _NB
    t0 = tr * _TBLK
    b0 = wb * _BBLK

    pltpu.sync_copy(xt_hbm.at[pl.ds(t0, _TBLK), pl.ds(b0, _BBLK)], idx_v)

    row_iotas = [lax.iota(jnp.int32, 16) + 16 * g for g in range(8)]

    def gather_start(ti, c, b):
        pltpu.async_copy(
            table_hbm.at[idx_v.at[ti, pl.ds(c * _CHUNK, _CHUNK)]],
            rows_v.at[b], gsem.at[b]
        )

    def gather_wait(b):
        pltpu.make_async_copy(
            table_hbm.at[pl.ds(0, _CHUNK)], rows_v.at[b], gsem.at[b]
        ).wait()

    def out_start(ti, tb):
        pltpu.async_copy(
            trows_v.at[tb],
            out_hbm.at[t0 + ti, :, pl.ds(b0, _BBLK)],
            osem.at[tb],
        )

    def out_wait(tb):
        pltpu.make_async_copy(
            trows_v.at[tb],
            out_hbm.at[0, :, pl.ds(0, _BBLK)],
            osem.at[tb],
        ).wait()

    def transpose_chunk(b, tb, c):
        @plsc.parallel_loop(0, INPUT_DIM, 1, unroll=8)
        def _(d):
            col = jnp.full((16,), jnp.int32(0)) + d
            for gi in range(8):
                v = plsc.load_gather(rows_v.at[b], [row_iotas[gi], col])
                trows_v[tb, d, pl.ds(_CHUNK * c + 16 * gi, 16)] = v

    gather_start(0, 0, 0)
    gather_start(0, 1, 1)

    def body(ti, carry):
        tb = lax.rem(ti, 2)

        @pl.when(ti > 1)
        def _():
            out_wait(tb)

        for c in range(_NC):
            b = c % _NBUF
            gather_wait(b)
            transpose_chunk(b, tb, c)
            if c + _NBUF < _NC:
                gather_start(ti, c + _NBUF, b)
            else:
                @pl.when(ti + 1 < _TBLK)
                def _():
                    gather_start(ti + 1, c + _NBUF - _NC, b)

        out_start(ti, tb)
        return carry

    lax.fori_loop(0, _TBLK, body, 0)

    for tb in range(2):
        out_wait(tb)


@jax.jit
def kernel(x, emb_table):
    xt = x.T
    run = functools.partial(
        pl.kernel,
        mesh=plsc.VectorSubcoreMesh(core_axis_name="c", subcore_axis_name="s"),
        out_type=jax.ShapeDtypeStruct((HIST, INPUT_DIM, BATCH), jnp.float32),
        scratch_types=[
            pltpu.VMEM((_TBLK, _BBLK), jnp.int32),
            pltpu.VMEM((_NBUF, _CHUNK, INPUT_DIM), jnp.float32),
            pltpu.VMEM((2, INPUT_DIM, _BBLK), jnp.float32),
            pltpu.SemaphoreType.DMA((_NBUF,)),
            pltpu.SemaphoreType.DMA((2,)),
        ],
        compiler_params=pltpu.CompilerParams(use_tc_tiling_on_sc=False,
                                             needs_layout_passes=False),
    )(_gather_kernel)
    out_t = run(emb_table, xt)
    return jnp.transpose(out_t, (2, 0, 1))


# speedup vs baseline: 1.3504x; 1.3473x over previous
import functools

import jax
import jax.numpy as jnp
from jax import lax
from jax.experimental import pallas as pl
from jax.experimental.pallas import tpu as pltpu
from jax.experimental.pallas import tpu_sc as plsc

VOCAB = 1000000
INPUT_DIM = 64
BATCH = 4096
HIST = 200

_BBLK = 512
_NB = BATCH // _BBLK
_TBLK = 50
_CHUNK = 128
_NC = _BBLK // _CHUNK
_NBUF = 2


def _gather_kernel(table_hbm, xt_hbm, out_hbm, idx_v, rows_v, trows_v,
                   gsem, osem):
    nc = 2
    wid = lax.axis_index("s") * nc + lax.axis_index("c")
    tr = wid // _NB
    wb = wid % _NB
    t0 = tr * _TBLK
    b0 = wb * _BBLK

    pltpu.sync_copy(xt_hbm.at[pl.ds(t0, _TBLK), pl.ds(b0, _BBLK)], idx_v)

    row_iotas = [lax.iota(jnp.int32, 16) + 16 * g for g in range(8)]

    def gather_start(ti, c, b):
        pltpu.async_copy(
            table_hbm.at[idx_v.at[ti, pl.ds(c * _CHUNK, _CHUNK)]],
            rows_v.at[b], gsem.at[b]
        )

    def gather_wait(b):
        pltpu.make_async_copy(
            table_hbm.at[pl.ds(0, _CHUNK)], rows_v.at[b], gsem.at[b]
        ).wait()

    def out_start(ti, tb):
        pltpu.async_copy(
            trows_v.at[tb],
            out_hbm.at[t0 + ti, :, pl.ds(b0, _BBLK)],
            osem.at[tb],
        )

    def out_wait(tb):
        pltpu.make_async_copy(
            trows_v.at[tb],
            out_hbm.at[0, :, pl.ds(0, _BBLK)],
            osem.at[tb],
        ).wait()

    def transpose_chunk(b, tb, c):
        bcols = [row_iotas[kb] + _CHUNK * c for kb in range(8)]

        @plsc.parallel_loop(0, 16, 1, unroll=2)
        def _(r):
            rr = lax.rem(lax.iota(jnp.int32, 16) + r, 16)
            for kb in range(8):
                for db in range(4):
                    col = rr + 16 * db
                    v = plsc.load_gather(rows_v.at[b],
                                         [row_iotas[kb], col])
                    plsc.store_scatter(trows_v.at[tb], [col, bcols[kb]], v)

    gather_start(0, 0, 0)
    gather_start(0, 1, 1)

    def body(ti, carry):
        tb = lax.rem(ti, 2)

        @pl.when(ti > 1)
        def _():
            out_wait(tb)

        for c in range(_NC):
            b = c % _NBUF
            gather_wait(b)
            transpose_chunk(b, tb, c)
            if c + _NBUF < _NC:
                gather_start(ti, c + _NBUF, b)
            else:
                @pl.when(ti + 1 < _TBLK)
                def _():
                    gather_start(ti + 1, c + _NBUF - _NC, b)

        out_start(ti, tb)
        return carry

    lax.fori_loop(0, _TBLK, body, 0)

    for tb in range(2):
        out_wait(tb)


@jax.jit
def kernel(x, emb_table):
    xt = x.T
    run = functools.partial(
        pl.kernel,
        mesh=plsc.VectorSubcoreMesh(core_axis_name="c", subcore_axis_name="s"),
        out_type=jax.ShapeDtypeStruct((HIST, INPUT_DIM, BATCH), jnp.float32),
        scratch_types=[
            pltpu.VMEM((_TBLK, _BBLK), jnp.int32),
            pltpu.VMEM((_NBUF, _CHUNK, INPUT_DIM), jnp.float32),
            pltpu.VMEM((2, INPUT_DIM, _BBLK), jnp.float32),
            pltpu.SemaphoreType.DMA((_NBUF,)),
            pltpu.SemaphoreType.DMA((2,)),
        ],
        compiler_params=pltpu.CompilerParams(use_tc_tiling_on_sc=False,
                                             needs_layout_passes=False),
    )(_gather_kernel)
    out_t = run(emb_table, xt)
    return jnp.transpose(out_t, (2, 0, 1))


# speedup vs baseline: 1.3905x; 1.0297x over previous
import functools

import jax
import jax.numpy as jnp
from jax import lax
from jax.experimental import pallas as pl
from jax.experimental.pallas import tpu as pltpu
from jax.experimental.pallas import tpu_sc as plsc

VOCAB = 1000000
INPUT_DIM = 64
BATCH = 4096
HIST = 200

_BBLK = 512
_NB = BATCH // _BBLK
_TBLK = 50
_CHUNK = 128
_NC = _BBLK // _CHUNK
_NBUF = 4


def _gather_kernel(table_hbm, xt_hbm, out_hbm, idx_v, rows_v, trows_v,
                   gsem, osem):
    nc = 2
    wid = lax.axis_index("s") * nc + lax.axis_index("c")
    tr = wid // _NB
    wb = wid % _NB
    t0 = tr * _TBLK
    b0 = wb * _BBLK

    pltpu.sync_copy(xt_hbm.at[pl.ds(t0, _TBLK), pl.ds(b0, _BBLK)], idx_v)

    row_iotas = [lax.iota(jnp.int32, 16) + 16 * g for g in range(8)]

    def gather_start(ti, c, b):
        pltpu.async_copy(
            table_hbm.at[idx_v.at[ti, pl.ds(c * _CHUNK, _CHUNK)]],
            rows_v.at[b], gsem.at[b]
        )

    def gather_wait(b):
        pltpu.make_async_copy(
            table_hbm.at[pl.ds(0, _CHUNK)], rows_v.at[b], gsem.at[b]
        ).wait()

    def out_start(ti, tb):
        pltpu.async_copy(
            trows_v.at[tb],
            out_hbm.at[t0 + ti, :, pl.ds(b0, _BBLK)],
            osem.at[tb],
        )

    def out_wait(tb):
        pltpu.make_async_copy(
            trows_v.at[tb],
            out_hbm.at[0, :, pl.ds(0, _BBLK)],
            osem.at[tb],
        ).wait()

    def transpose_chunk(b, tb, c):
        bcols = [row_iotas[kb] + _CHUNK * c for kb in range(8)]

        @plsc.parallel_loop(0, 16, 1, unroll=2)
        def _(r):
            rr = lax.rem(lax.iota(jnp.int32, 16) + r, 16)
            for kb in range(8):
                for db in range(4):
                    col = rr + 16 * db
                    v = plsc.load_gather(rows_v.at[b],
                                         [row_iotas[kb], col])
                    plsc.store_scatter(trows_v.at[tb], [col, bcols[kb]], v)

    for c in range(_NBUF):
        gather_start(0, c, c)

    def body(ti, carry):
        tb = lax.rem(ti, 2)

        @pl.when(ti > 1)
        def _():
            out_wait(tb)

        for c in range(_NC):
            b = c % _NBUF
            gather_wait(b)
            transpose_chunk(b, tb, c)
            if c + _NBUF < _NC:
                gather_start(ti, c + _NBUF, b)
            else:
                @pl.when(ti + 1 < _TBLK)
                def _():
                    gather_start(ti + 1, c + _NBUF - _NC, b)

        out_start(ti, tb)
        return carry

    lax.fori_loop(0, _TBLK, body, 0)

    for tb in range(2):
        out_wait(tb)


@jax.jit
def kernel(x, emb_table):
    xt = x.T
    run = functools.partial(
        pl.kernel,
        mesh=plsc.VectorSubcoreMesh(core_axis_name="c", subcore_axis_name="s"),
        out_type=jax.ShapeDtypeStruct((HIST, INPUT_DIM, BATCH), jnp.float32),
        scratch_types=[
            pltpu.VMEM((_TBLK, _BBLK), jnp.int32),
            pltpu.VMEM((_NBUF, _CHUNK, INPUT_DIM), jnp.float32),
            pltpu.VMEM((2, INPUT_DIM, _BBLK), jnp.float32),
            pltpu.SemaphoreType.DMA((_NBUF,)),
            pltpu.SemaphoreType.DMA((2,)),
        ],
        compiler_params=pltpu.CompilerParams(use_tc_tiling_on_sc=False,
                                             needs_layout_passes=False),
    )(_gather_kernel)
    out_t = run(emb_table, xt)
    return jnp.transpose(out_t, (2, 0, 1))
